# Initial kernel scaffold; baseline (speedup 1.0000x reference)
import jax
import jax.numpy as jnp
from jax.experimental import pallas as pl


def kernel(x, edge_index, edge_type, bases, coefficients, self_loop):
    raise NotImplementedError("write your pallas kernel here")


# speedup vs baseline: 59.8738x; 59.8738x over previous
import functools

import jax
import jax.numpy as jnp
from jax import lax
from jax.experimental import pallas as pl
from jax.experimental.pallas import tpu as pltpu
from jax.experimental.pallas import tpu_sc as plsc

N = 10000
D = 128
R = 8
E = 320000
FLAT = R * N

NC = 2
NS = 16
CH = 80
STAGE = 25
SE = STAGE * CH
DEG_PAD = NS * 5120

E_PER_SC = E // NC
E2_PER_TILE = E_PER_SC // NS
E1_PER_TILE = E // NS

NROW = 624


def _y_body(coeff_smem, bases_ref, x_ref, y_ref):
    r = pl.program_id(0)
    w = coeff_smem[r, 0] * bases_ref[0]
    for b in range(1, 4):
        w = w + coeff_smem[r, b] * bases_ref[b]
    y_ref[0] = jnp.dot(x_ref[...], w, preferred_element_type=jnp.float32)


def _make_y(coefficients, bases, x):
    return pl.pallas_call(
        _y_body,
        grid=(R,),
        in_specs=[
            pl.BlockSpec(memory_space=pltpu.SMEM),
            pl.BlockSpec((4, D, D), lambda r: (0, 0, 0)),
            pl.BlockSpec((N, D), lambda r: (0, 0)),
        ],
        out_specs=pl.BlockSpec((1, N, D), lambda r: (r, 0, 0)),
        out_shape=jax.ShapeDtypeStruct((R, N, D), jnp.float32),
    )(coefficients, bases, x)


def _out_body(x_ref, sl_ref, a0_ref, a1_ref, o_ref):
    o_ref[...] = (
        jnp.dot(x_ref[...], sl_ref[...], preferred_element_type=jnp.float32)
        + a0_ref[...] + a1_ref[...]
    )


def _make_out(x, self_loop, a0, a1):
    blk = 2000
    return pl.pallas_call(
        _out_body,
        grid=(N // blk,),
        in_specs=[
            pl.BlockSpec((blk, D), lambda m: (m, 0)),
            pl.BlockSpec((D, D), lambda m: (0, 0)),
            pl.BlockSpec((blk, D), lambda m: (m, 0)),
            pl.BlockSpec((blk, D), lambda m: (m, 0)),
        ],
        out_specs=pl.BlockSpec((blk, D), lambda m: (m, 0)),
        out_shape=jax.ShapeDtypeStruct((N, D), jnp.float32),
    )(x, self_loop, a0, a1)


def _sc_body(y_hbm, src_hbm, dst_hbm, typ_hbm, out_hbm,
             deg_sh, acc_sh,
             zrows, zbuf, sstage, dstage, tstage,
             rows0, g0, f0, d0, cbuf, ones, fd,
             sem0):
    c = lax.axis_index("c")
    s = lax.axis_index("s")

    zero16 = jnp.zeros((16,), jnp.float32)

    def _z1(i, _):
        zbuf[pl.ds(i * 16, 16)] = zero16
        return 0
    lax.fori_loop(0, 320, _z1, 0)

    def _z2(i, _):
        r = i // 8
        col = (i % 8) * 16
        zrows[r, pl.ds(col, 16)] = zero16
        return 0
    lax.fori_loop(0, 104 * 8, _z2, 0)

    pltpu.sync_copy(zbuf, deg_sh.at[pl.ds(s * 5120, 5120)])
    for q in range(6):
        pltpu.sync_copy(zrows, acc_sh.at[pl.ds(s * NROW + q * 104, 104)])

    @pl.when(s == 15)
    def _():
        pltpu.sync_copy(zrows.at[pl.ds(0, 16)], acc_sh.at[pl.ds(9984, 16)])

    for i in range(5):
        ones[pl.ds(i * 16, 16)] = jnp.ones((16,), jnp.float32)

    plsc.subcore_barrier()

    ten_k = jnp.int32(N)

    def _deg_stage(b, _):
        e0 = s * E1_PER_TILE + b * SE
        pltpu.sync_copy(dst_hbm.at[pl.ds(e0, SE)], dstage)
        pltpu.sync_copy(typ_hbm.at[pl.ds(e0, SE)], tstage)

        def _deg_chunk(k, _):
            def _mk(i, _):
                t = tstage[pl.ds(k * CH + i * 16, 16)]
                dd = dstage[pl.ds(k * CH + i * 16, 16)]
                fd[pl.ds(i * 16, 16)] = t * ten_k + dd
                return 0
            lax.fori_loop(0, CH // 16, _mk, 0)
            pltpu.sync_copy(ones, deg_sh.at[fd], add=True)
            return 0
        lax.fori_loop(0, STAGE, _deg_chunk, 0)
        return 0
    lax.fori_loop(0, E1_PER_TILE // SE, _deg_stage, 0)

    plsc.subcore_barrier()

    pltpu.sync_copy(deg_sh.at[pl.ds(s * 5120, 5120)], zbuf)

    def _recip(i, _):
        v = zbuf[pl.ds(i * 16, 16)]
        zbuf[pl.ds(i * 16, 16)] = 1.0 / jnp.maximum(v, 1.0)
        return 0
    lax.fori_loop(0, 320, _recip, 0)
    pltpu.sync_copy(zbuf, deg_sh.at[pl.ds(s * 5120, 5120)])

    plsc.subcore_barrier()

    tile_e0 = (c * NS + s) * E2_PER_TILE

    def _main_stage(b, _):
        e0 = tile_e0 + b * SE
        pltpu.sync_copy(src_hbm.at[pl.ds(e0, SE)], sstage)
        pltpu.sync_copy(dst_hbm.at[pl.ds(e0, SE)], dstage)
        pltpu.sync_copy(typ_hbm.at[pl.ds(e0, SE)], tstage)

        def _chunk(k, _):
            def _mk(i, _):
                t = tstage[pl.ds(k * CH + i * 16, 16)]
                ss = sstage[pl.ds(k * CH + i * 16, 16)]
                dd = dstage[pl.ds(k * CH + i * 16, 16)]
                tn = t * ten_k
                g0[pl.ds(i * 16, 16)] = tn + ss
                f0[pl.ds(i * 16, 16)] = tn + dd
                d0[pl.ds(i * 16, 16)] = dd
                return 0
            lax.fori_loop(0, CH // 16, _mk, 0)
            pltpu.async_copy(y_hbm.at[g0], rows0, sem0).wait()
            pltpu.sync_copy(deg_sh.at[f0], cbuf)

            def _mul(g, _):
                cv = cbuf[pl.ds(g * 16, 16)]
                for l in range(16):
                    cs = cv[l]
                    e = g * 16 + l
                    for j in range(8):
                        rows0[e, pl.ds(j * 16, 16)] = (
                            rows0[e, pl.ds(j * 16, 16)] * cs)
                return 0
            lax.fori_loop(0, CH // 16, _mul, 0)
            pltpu.sync_copy(rows0, acc_sh.at[d0], add=True)
            return 0
        lax.fori_loop(0, STAGE, _chunk, 0)
        return 0
    lax.fori_loop(0, E2_PER_TILE // SE, _main_stage, 0)

    plsc.subcore_barrier()

    pltpu.sync_copy(acc_sh.at[pl.ds(s * NROW, NROW)],
                    out_hbm.at[c, pl.ds(s * NROW, NROW)])

    @pl.when(s == 15)
    def _():
        pltpu.sync_copy(acc_sh.at[pl.ds(9984, 16)],
                        out_hbm.at[c, pl.ds(9984, 16)])


def _make_sc(y_flat, src, dst, typ):
    mesh = plsc.VectorSubcoreMesh(core_axis_name="c", subcore_axis_name="s")
    run = functools.partial(
        pl.kernel,
        out_type=jax.ShapeDtypeStruct((NC, N, D), jnp.float32),
        mesh=mesh,
        scratch_types=[
            pltpu.VMEM_SHARED((DEG_PAD,), jnp.float32),
            pltpu.VMEM_SHARED((N, D), jnp.float32),
            pltpu.VMEM((104, D), jnp.float32),
            pltpu.VMEM((5120,), jnp.float32),
            pltpu.VMEM((SE,), jnp.int32),
            pltpu.VMEM((SE,), jnp.int32),
            pltpu.VMEM((SE,), jnp.int32),
            pltpu.VMEM((CH, D), jnp.float32),
            pltpu.VMEM((CH,), jnp.int32),
            pltpu.VMEM((CH,), jnp.int32),
            pltpu.VMEM((CH,), jnp.int32),
            pltpu.VMEM((CH,), jnp.float32),
            pltpu.VMEM((CH,), jnp.float32),
            pltpu.VMEM((CH,), jnp.int32),
            pltpu.SemaphoreType.DMA,
        ],
    )(_sc_body)
    return run(y_flat, src, dst, typ)


def kernel(x, edge_index, edge_type, bases, coefficients, self_loop):
    src = edge_index[0].astype(jnp.int32)
    dst = edge_index[1].astype(jnp.int32)
    typ = edge_type.astype(jnp.int32)

    y = _make_y(coefficients, bases, x).reshape(FLAT, D)
    acc = _make_sc(y, src, dst, typ)
    return _make_out(x, self_loop, acc[0], acc[1])


# speedup vs baseline: 93.3301x; 1.5588x over previous
import functools

import jax
import jax.numpy as jnp
from jax import lax
from jax.experimental import pallas as pl
from jax.experimental.pallas import tpu as pltpu
from jax.experimental.pallas import tpu_sc as plsc

N = 10000
D = 128
R = 8
E = 320000
FLAT = R * N

NC = 2
NS = 16
CH = 80
STAGE = 25
SE = STAGE * CH
DEG_PAD = NS * 5120

E_PER_SC = E // NC
E2_PER_TILE = E_PER_SC // NS
E1_PER_TILE = E // NS

NROW = 624


def _y_body(coeff_smem, bases_ref, x_ref, y_ref):
    r = pl.program_id(0)
    w = coeff_smem[r, 0] * bases_ref[0]
    for b in range(1, 4):
        w = w + coeff_smem[r, b] * bases_ref[b]
    y_ref[0] = jnp.dot(x_ref[...], w, preferred_element_type=jnp.float32)


def _make_y(coefficients, bases, x):
    return pl.pallas_call(
        _y_body,
        grid=(R,),
        in_specs=[
            pl.BlockSpec(memory_space=pltpu.SMEM),
            pl.BlockSpec((4, D, D), lambda r: (0, 0, 0)),
            pl.BlockSpec((N, D), lambda r: (0, 0)),
        ],
        out_specs=pl.BlockSpec((1, N, D), lambda r: (r, 0, 0)),
        out_shape=jax.ShapeDtypeStruct((R, N, D), jnp.float32),
    )(coefficients, bases, x)


def _out_body(x_ref, sl_ref, a0_ref, a1_ref, o_ref):
    o_ref[...] = (
        jnp.dot(x_ref[...], sl_ref[...], preferred_element_type=jnp.float32)
        + a0_ref[...] + a1_ref[...]
    )


def _make_out(x, self_loop, a0, a1):
    blk = 2000
    return pl.pallas_call(
        _out_body,
        grid=(N // blk,),
        in_specs=[
            pl.BlockSpec((blk, D), lambda m: (m, 0)),
            pl.BlockSpec((D, D), lambda m: (0, 0)),
            pl.BlockSpec((blk, D), lambda m: (m, 0)),
            pl.BlockSpec((blk, D), lambda m: (m, 0)),
        ],
        out_specs=pl.BlockSpec((blk, D), lambda m: (m, 0)),
        out_shape=jax.ShapeDtypeStruct((N, D), jnp.float32),
    )(x, self_loop, a0, a1)


def _sc_body(y_hbm, src_hbm, dst_hbm, typ_hbm, out_hbm,
             deg_sh, acc_sh,
             zrows, zbuf, sstage, dstage, tstage, dscat,
             rows0, rows1, g0, g1, f0, f1, cb0, cb1, ones,
             fd0, fd1, fd2, fd3, fd4,
             semg0, semg1, semc0, semc1, semd):
    c = lax.axis_index("c")
    s = lax.axis_index("s")

    zero16 = jnp.zeros((16,), jnp.float32)

    def _z1(i, _):
        zbuf[pl.ds(i * 16, 16)] = zero16
        return 0
    lax.fori_loop(0, 320, _z1, 0)

    def _z2(i, _):
        zrows[i // 8, pl.ds((i % 8) * 16, 16)] = zero16
        return 0
    lax.fori_loop(0, 16 * 8, _z2, 0)

    pltpu.sync_copy(zbuf, deg_sh.at[pl.ds(s * 5120, 5120)])
    for q in range(NROW // 16):
        pltpu.sync_copy(zrows, acc_sh.at[pl.ds(s * NROW + q * 16, 16)])

    @pl.when(s == 15)
    def _():
        pltpu.sync_copy(zrows, acc_sh.at[pl.ds(9984, 16)])

    for i in range(5):
        ones[pl.ds(i * 16, 16)] = jnp.ones((16,), jnp.float32)

    plsc.subcore_barrier()

    ten_k = jnp.int32(N)
    fds = (fd0, fd1, fd2, fd3, fd4)

    def _deg_block(b, _):
        e0 = s * E1_PER_TILE + b * SE
        pltpu.sync_copy(dst_hbm.at[pl.ds(e0, SE)], dstage)
        pltpu.sync_copy(typ_hbm.at[pl.ds(e0, SE)], tstage)

        def _deg_group(gi, _):
            descs = []
            for j in range(5):
                k = gi * 5 + j
                def _mk(i, _, k=k, fd=fds[j]):
                    t = tstage[pl.ds(k * CH + i * 16, 16)]
                    dd = dstage[pl.ds(k * CH + i * 16, 16)]
                    fd[pl.ds(i * 16, 16)] = t * ten_k + dd
                    return 0
                lax.fori_loop(0, CH // 16, _mk, 0)
                descs.append(
                    pltpu.async_copy(ones, deg_sh.at[fds[j]], semd, add=True))
            for dsc in descs:
                dsc.wait()
            return 0
        lax.fori_loop(0, STAGE // 5, _deg_group, 0)
        return 0
    lax.fori_loop(0, E1_PER_TILE // SE, _deg_block, 0)

    plsc.subcore_barrier()

    pltpu.sync_copy(deg_sh.at[pl.ds(s * 5120, 5120)], zbuf)

    def _recip(i, _):
        v = zbuf[pl.ds(i * 16, 16)]
        zbuf[pl.ds(i * 16, 16)] = 1.0 / jnp.maximum(v, 1.0)
        return 0
    lax.fori_loop(0, 320, _recip, 0)
    pltpu.sync_copy(zbuf, deg_sh.at[pl.ds(s * 5120, 5120)])

    plsc.subcore_barrier()

    tile_e0 = (c * NS + s) * E2_PER_TILE

    def _bld(k, g, f):
        def _mk(i, _):
            t = tstage[pl.ds(k * CH + i * 16, 16)]
            ss = sstage[pl.ds(k * CH + i * 16, 16)]
            dd = dstage[pl.ds(k * CH + i * 16, 16)]
            tn = t * ten_k
            g[pl.ds(i * 16, 16)] = tn + ss
            f[pl.ds(i * 16, 16)] = tn + dd
            return 0
        lax.fori_loop(0, CH // 16, _mk, 0)

    def _fire(k, g, f, rows, cb, semg, semc):
        _bld(k, g, f)
        pltpu.async_copy(y_hbm.at[g], rows, semg)
        pltpu.async_copy(deg_sh.at[f], cb, semc)

    def _finish(k, g, f, rows, cb, semg, semc):
        pltpu.make_async_copy(y_hbm.at[g], rows, semg).wait()
        pltpu.make_async_copy(deg_sh.at[f], cb, semc).wait()

        def _mul(gg, _):
            cv = cb[pl.ds(gg * 16, 16)]
            for l in range(16):
                cs = cv[l]
                ei = gg * 16 + l
                for j in range(8):
                    rows[ei, pl.ds(j * 16, 16)] = (
                        rows[ei, pl.ds(j * 16, 16)] * cs)
            return 0
        lax.fori_loop(0, CH // 16, _mul, 0)
        pltpu.sync_copy(rows, acc_sh.at[dscat.at[k]], add=True)

    def _main_block(b, _):
        e0 = tile_e0 + b * SE
        pltpu.sync_copy(src_hbm.at[pl.ds(e0, SE)], sstage)
        pltpu.sync_copy(dst_hbm.at[pl.ds(e0, SE)], dstage)
        pltpu.sync_copy(typ_hbm.at[pl.ds(e0, SE)], tstage)

        def _dcp(i, _):
            dscat[i // 5, pl.ds((i % 5) * 16, 16)] = dstage[pl.ds(i * 16, 16)]
            return 0
        lax.fori_loop(0, STAGE * 5, _dcp, 0)

        _fire(0, g0, f0, rows0, cb0, semg0, semc0)

        def _pair(m, _):
            a = 2 * m
            _fire(a + 1, g1, f1, rows1, cb1, semg1, semc1)
            _finish(a, g0, f0, rows0, cb0, semg0, semc0)
            _fire(a + 2, g0, f0, rows0, cb0, semg0, semc0)
            _finish(a + 1, g1, f1, rows1, cb1, semg1, semc1)
            return 0
        lax.fori_loop(0, (STAGE - 1) // 2, _pair, 0)
        _finish(STAGE - 1, g0, f0, rows0, cb0, semg0, semc0)
        return 0
    lax.fori_loop(0, E2_PER_TILE // SE, _main_block, 0)

    plsc.subcore_barrier()

    pltpu.sync_copy(acc_sh.at[pl.ds(s * NROW, NROW)],
                    out_hbm.at[c, pl.ds(s * NROW, NROW)])

    @pl.when(s == 15)
    def _():
        pltpu.sync_copy(acc_sh.at[pl.ds(9984, 16)],
                        out_hbm.at[c, pl.ds(9984, 16)])


def _make_sc(y_flat, src, dst, typ):
    mesh = plsc.VectorSubcoreMesh(core_axis_name="c", subcore_axis_name="s")
    run = functools.partial(
        pl.kernel,
        out_type=jax.ShapeDtypeStruct((NC, N, D), jnp.float32),
        mesh=mesh,
        scratch_types=[
            pltpu.VMEM_SHARED((DEG_PAD,), jnp.float32),
            pltpu.VMEM_SHARED((N, D), jnp.float32),
            pltpu.VMEM((16, D), jnp.float32),
            pltpu.VMEM((5120,), jnp.float32),
            pltpu.VMEM((SE,), jnp.int32),
            pltpu.VMEM((SE,), jnp.int32),
            pltpu.VMEM((SE,), jnp.int32),
            pltpu.VMEM((STAGE, CH), jnp.int32),
            pltpu.VMEM((CH, D), jnp.float32),
            pltpu.VMEM((CH, D), jnp.float32),
            pltpu.VMEM((CH,), jnp.int32),
            pltpu.VMEM((CH,), jnp.int32),
            pltpu.VMEM((CH,), jnp.int32),
            pltpu.VMEM((CH,), jnp.int32),
            pltpu.VMEM((CH,), jnp.float32),
            pltpu.VMEM((CH,), jnp.float32),
            pltpu.VMEM((CH,), jnp.float32),
            pltpu.VMEM((CH,), jnp.int32),
            pltpu.VMEM((CH,), jnp.int32),
            pltpu.VMEM((CH,), jnp.int32),
            pltpu.VMEM((CH,), jnp.int32),
            pltpu.VMEM((CH,), jnp.int32),
            pltpu.SemaphoreType.DMA,
            pltpu.SemaphoreType.DMA,
            pltpu.SemaphoreType.DMA,
            pltpu.SemaphoreType.DMA,
            pltpu.SemaphoreType.DMA,
        ],
    )(_sc_body)
    return run(y_flat, src, dst, typ)


def kernel(x, edge_index, edge_type, bases, coefficients, self_loop):
    src = edge_index[0].astype(jnp.int32)
    dst = edge_index[1].astype(jnp.int32)
    typ = edge_type.astype(jnp.int32)

    y = _make_y(coefficients, bases, x).reshape(FLAT, D)
    acc = _make_sc(y, src, dst, typ)
    return _make_out(x, self_loop, acc[0], acc[1])
